# baseline (speedup 1.0000x reference)
import functools

import jax
import jax.numpy as jnp
from jax import lax
from jax.experimental import pallas as pl
from jax.experimental.pallas import tpu as pltpu
from jax.experimental.pallas import tpu_sc as plsc

RX = RY = RZ = 32
R = RX * RY * RZ

NC, NS, L = 2, 16, 16
NW = NC * NS


def _coords_body(coords_ref, nc_ref, idx_ref):
    c = coords_ref[0]
    mean = jnp.mean(c, axis=1, keepdims=True)
    cc = c - mean
    norm = jnp.sqrt(jnp.sum(cc * cc, axis=0, keepdims=True))
    denom = jnp.max(norm) * 2.0
    s = jnp.clip((cc / denom + 0.5) * RX, 0, RX - 1)
    nc_ref[0] = s
    v = jnp.round(s).astype(jnp.int32)
    idx_ref[0, 0] = v[0] * (RY * RZ) + v[1] * RZ + v[2]


def _make_scatter(B, C, N, chunk):
    cpw = C // NW
    nchunks = N // chunk
    assert N == nchunks * chunk and chunk % L == 0
    mesh = plsc.VectorSubcoreMesh(
        core_axis_name="c", subcore_axis_name="s", num_cores=NC, num_subcores=NS)

    @functools.partial(
        pl.kernel,
        out_type=jax.ShapeDtypeStruct((B * C * R,), jnp.float32),
        mesh=mesh,
        compiler_params=pltpu.CompilerParams(needs_layout_passes=False),
        scratch_types=[
            pltpu.VMEM((cpw * R,), jnp.float32),
            pltpu.VMEM((R,), jnp.float32),
            pltpu.VMEM((2 * chunk,), jnp.int32),
            pltpu.VMEM((2 * chunk,), jnp.float32),
            pltpu.VMEM((2 * chunk,), jnp.float32),
            pltpu.SemaphoreType.DMA,
            pltpu.SemaphoreType.DMA,
        ],
    )
    def scatter(feat_hbm, idx_hbm, out_hbm, sums, cnts, idxb, v0b, v1b, sem0, sem1):
        wid = lax.axis_index("s") * NC + lax.axis_index("c")
        c0 = wid * cpw
        zero = jnp.zeros((L,), jnp.float32)
        ones = jnp.ones((L,), jnp.float32)
        roff = jnp.full((L,), R, jnp.int32)
        sems = (sem0, sem1)

        for b in range(B):
            @plsc.parallel_loop(0, (cpw * R) // L, unroll=8)
            def zsums(i):
                sums[pl.ds(i * L, L)] = zero

            @plsc.parallel_loop(0, R // L, unroll=8)
            def zcnts(i):
                cnts[pl.ds(i * L, L)] = zero

            def issue(k, slot):
                so = slot * chunk
                pltpu.async_copy(
                    idx_hbm.at[pl.ds(b * N + k * chunk, chunk)],
                    idxb.at[pl.ds(so, chunk)], sems[slot])
                pltpu.async_copy(
                    feat_hbm.at[pl.ds((b * C + c0) * N + k * chunk, chunk)],
                    v0b.at[pl.ds(so, chunk)], sems[slot])
                pltpu.async_copy(
                    feat_hbm.at[pl.ds((b * C + c0 + 1) * N + k * chunk, chunk)],
                    v1b.at[pl.ds(so, chunk)], sems[slot])

            def drain(k, slot):
                so = slot * chunk
                pltpu.make_async_copy(
                    idx_hbm.at[pl.ds(b * N + k * chunk, chunk)],
                    idxb.at[pl.ds(so, chunk)], sems[slot]).wait()
                pltpu.make_async_copy(
                    feat_hbm.at[pl.ds((b * C + c0) * N + k * chunk, chunk)],
                    v0b.at[pl.ds(so, chunk)], sems[slot]).wait()
                pltpu.make_async_copy(
                    feat_hbm.at[pl.ds((b * C + c0 + 1) * N + k * chunk, chunk)],
                    v1b.at[pl.ds(so, chunk)], sems[slot]).wait()

            def consume(slot):
                so = slot * chunk

                @plsc.parallel_loop(0, chunk // L, unroll=8)
                def g(i):
                    iv = idxb[pl.ds(so + i * L, L)]
                    plsc.addupdate_scatter(sums, [iv], v0b[pl.ds(so + i * L, L)])
                    plsc.addupdate_scatter(sums, [iv + roff], v1b[pl.ds(so + i * L, L)])
                    plsc.addupdate_scatter(cnts, [iv], ones)

            issue(0, 0)

            def chunk_pair(j, _):
                k = 2 * j
                issue(k + 1, 1)
                drain(k, 0)
                consume(0)

                @pl.when(k + 2 < nchunks)
                def _():
                    issue(k + 2, 0)

                drain(k + 1, 1)
                consume(1)
                return 0

            lax.fori_loop(0, nchunks // 2, chunk_pair, 0)
            if nchunks % 2:
                drain(nchunks - 1, 0)
                consume(0)

            @plsc.parallel_loop(0, R // L, unroll=4)
            def div(i):
                cv = jnp.maximum(cnts[pl.ds(i * L, L)], 1.0)
                sums[pl.ds(i * L, L)] = sums[pl.ds(i * L, L)] / cv
                sums[pl.ds(R + i * L, L)] = sums[pl.ds(R + i * L, L)] / cv
            pltpu.sync_copy(sums.at[pl.ds(0, R)],
                            out_hbm.at[pl.ds((b * C + c0) * R, R)])
            pltpu.sync_copy(sums.at[pl.ds(R, R)],
                            out_hbm.at[pl.ds((b * C + c0 + 1) * R, R)])

    return scatter


def kernel(features, coords):
    B, C, N = features.shape
    nc_out, flat_idx = pl.pallas_call(
        _coords_body,
        grid=(B,),
        in_specs=[pl.BlockSpec((1, 3, N), lambda b: (b, 0, 0))],
        out_specs=[
            pl.BlockSpec((1, 3, N), lambda b: (b, 0, 0)),
            pl.BlockSpec((1, 1, N), lambda b: (b, 0, 0)),
        ],
        out_shape=[
            jax.ShapeDtypeStruct((B, 3, N), jnp.float32),
            jax.ShapeDtypeStruct((B, 1, N), jnp.int32),
        ],
    )(coords)
    flat_idx = flat_idx.reshape(B, N)

    scatter = _make_scatter(B, C, N, chunk=4000)
    out = scatter(features.reshape(B * C * N), flat_idx.reshape(B * N))
    return out.reshape(B, C, RX, RY, RZ), nc_out


# speedup vs baseline: 1.1590x; 1.1590x over previous
import functools

import jax
import jax.numpy as jnp
from jax import lax
from jax.experimental import pallas as pl
from jax.experimental.pallas import tpu as pltpu
from jax.experimental.pallas import tpu_sc as plsc

RX = RY = RZ = 32
R = RX * RY * RZ

NC, NS, L = 2, 16, 16
NW = NC * NS


def _coords_body(coords_ref, nc_ref, idx_ref):
    c = coords_ref[0]
    mean = jnp.mean(c, axis=1, keepdims=True)
    cc = c - mean
    norm = jnp.sqrt(jnp.sum(cc * cc, axis=0, keepdims=True))
    denom = jnp.max(norm) * 2.0
    s = jnp.clip((cc / denom + 0.5) * RX, 0, RX - 1)
    nc_ref[0] = s
    v = jnp.round(s).astype(jnp.int32)
    idx_ref[0, 0] = v[0] * (RY * RZ) + v[1] * RZ + v[2]


def _make_scatter(B, C, N, chunk):
    cpw = C // NW
    nchunks = N // chunk
    assert N == nchunks * chunk and chunk % L == 0
    mesh = plsc.VectorSubcoreMesh(
        core_axis_name="c", subcore_axis_name="s", num_cores=NC, num_subcores=NS)

    @functools.partial(
        pl.kernel,
        out_type=jax.ShapeDtypeStruct((B, C, R), jnp.float32),
        mesh=mesh,
        compiler_params=pltpu.CompilerParams(needs_layout_passes=False),
        scratch_types=[
            pltpu.VMEM((cpw * R,), jnp.float32),
            pltpu.VMEM((R,), jnp.float32),
            pltpu.VMEM((2 * chunk,), jnp.int32),
            pltpu.VMEM((2 * chunk,), jnp.float32),
            pltpu.VMEM((2 * chunk,), jnp.float32),
            pltpu.SemaphoreType.DMA,
            pltpu.SemaphoreType.DMA,
        ],
    )
    def scatter(feat_hbm, idx_hbm, out_hbm, sums, cnts, idxb, v0b, v1b, sem0, sem1):
        wid = lax.axis_index("s") * NC + lax.axis_index("c")
        c0 = wid * cpw
        zero = jnp.zeros((L,), jnp.float32)
        ones = jnp.ones((L,), jnp.float32)
        roff = jnp.full((L,), R, jnp.int32)
        sems = (sem0, sem1)

        for b in range(B):
            @plsc.parallel_loop(0, (cpw * R) // L, unroll=8)
            def zsums(i):
                sums[pl.ds(i * L, L)] = zero

            @plsc.parallel_loop(0, R // L, unroll=8)
            def zcnts(i):
                cnts[pl.ds(i * L, L)] = zero

            def issue(k, slot):
                so = slot * chunk
                pltpu.async_copy(
                    idx_hbm.at[pl.ds(b * N + k * chunk, chunk)],
                    idxb.at[pl.ds(so, chunk)], sems[slot])
                pltpu.async_copy(
                    feat_hbm.at[pl.ds((b * C + c0) * N + k * chunk, chunk)],
                    v0b.at[pl.ds(so, chunk)], sems[slot])
                pltpu.async_copy(
                    feat_hbm.at[pl.ds((b * C + c0 + 1) * N + k * chunk, chunk)],
                    v1b.at[pl.ds(so, chunk)], sems[slot])

            def drain(k, slot):
                so = slot * chunk
                pltpu.make_async_copy(
                    idx_hbm.at[pl.ds(b * N + k * chunk, chunk)],
                    idxb.at[pl.ds(so, chunk)], sems[slot]).wait()
                pltpu.make_async_copy(
                    feat_hbm.at[pl.ds((b * C + c0) * N + k * chunk, chunk)],
                    v0b.at[pl.ds(so, chunk)], sems[slot]).wait()
                pltpu.make_async_copy(
                    feat_hbm.at[pl.ds((b * C + c0 + 1) * N + k * chunk, chunk)],
                    v1b.at[pl.ds(so, chunk)], sems[slot]).wait()

            def consume(slot):
                so = slot * chunk

                @plsc.parallel_loop(0, chunk // L, unroll=8)
                def g(i):
                    iv = idxb[pl.ds(so + i * L, L)]
                    plsc.addupdate_scatter(sums, [iv], v0b[pl.ds(so + i * L, L)])
                    plsc.addupdate_scatter(sums, [iv + roff], v1b[pl.ds(so + i * L, L)])
                    plsc.addupdate_scatter(cnts, [iv], ones)

            issue(0, 0)

            def chunk_pair(j, _):
                k = 2 * j
                issue(k + 1, 1)
                drain(k, 0)
                consume(0)

                @pl.when(k + 2 < nchunks)
                def _():
                    issue(k + 2, 0)

                drain(k + 1, 1)
                consume(1)
                return 0

            lax.fori_loop(0, nchunks // 2, chunk_pair, 0)
            if nchunks % 2:
                drain(nchunks - 1, 0)
                consume(0)

            @plsc.parallel_loop(0, R // L, unroll=4)
            def div(i):
                cv = jnp.maximum(cnts[pl.ds(i * L, L)], 1.0)
                sums[pl.ds(i * L, L)] = sums[pl.ds(i * L, L)] / cv
                sums[pl.ds(R + i * L, L)] = sums[pl.ds(R + i * L, L)] / cv
            pltpu.sync_copy(sums.at[pl.ds(0, R)], out_hbm.at[b, c0])
            pltpu.sync_copy(sums.at[pl.ds(R, R)], out_hbm.at[b, c0 + 1])

    return scatter


def kernel(features, coords):
    B, C, N = features.shape
    nc_out, flat_idx = pl.pallas_call(
        _coords_body,
        grid=(B,),
        in_specs=[pl.BlockSpec((1, 3, N), lambda b: (b, 0, 0))],
        out_specs=[
            pl.BlockSpec((1, 3, N), lambda b: (b, 0, 0)),
            pl.BlockSpec((1, 1, N), lambda b: (b, 0, 0)),
        ],
        out_shape=[
            jax.ShapeDtypeStruct((B, 3, N), jnp.float32),
            jax.ShapeDtypeStruct((B, 1, N), jnp.int32),
        ],
    )(coords)
    flat_idx = flat_idx.reshape(B, N)

    scatter = _make_scatter(B, C, N, chunk=4000)
    out = scatter(features.reshape(B * C * N), flat_idx.reshape(B * N))
    return out.reshape(B, C, RX, RY, RZ), nc_out


# speedup vs baseline: 1.1893x; 1.0261x over previous
import functools

import jax
import jax.numpy as jnp
from jax import lax
from jax.experimental import pallas as pl
from jax.experimental.pallas import tpu as pltpu
from jax.experimental.pallas import tpu_sc as plsc

RX = RY = RZ = 32
R = RX * RY * RZ

NC, NS, L = 2, 16, 16
NW = NC * NS


def _coords_body(coords_ref, nc_ref, idx_ref):
    c = coords_ref[0]
    mean = jnp.mean(c, axis=1, keepdims=True)
    cc = c - mean
    norm = jnp.sqrt(jnp.sum(cc * cc, axis=0, keepdims=True))
    denom = jnp.max(norm) * 2.0
    s = jnp.clip((cc / denom + 0.5) * RX, 0, RX - 1)
    nc_ref[0] = s
    v = jnp.round(s).astype(jnp.int32)
    idx_ref[0, 0] = v[0] * (RY * RZ) + v[1] * RZ + v[2]


def _make_scatter(B, C, N, chunk):
    cpw = C // NW
    nchunks = N // chunk
    assert N == nchunks * chunk and chunk % L == 0
    mesh = plsc.VectorSubcoreMesh(
        core_axis_name="c", subcore_axis_name="s", num_cores=NC, num_subcores=NS)

    @functools.partial(
        pl.kernel,
        out_type=[
            jax.ShapeDtypeStruct((B * C * R,), jnp.float32),
            jax.ShapeDtypeStruct((B * R,), jnp.float32),
        ],
        mesh=mesh,
        compiler_params=pltpu.CompilerParams(needs_layout_passes=False),
        scratch_types=[
            pltpu.VMEM((cpw * R,), jnp.float32),
            pltpu.VMEM((R,), jnp.float32),
            pltpu.VMEM((2 * chunk,), jnp.int32),
            pltpu.VMEM((2 * chunk,), jnp.float32),
            pltpu.VMEM((2 * chunk,), jnp.float32),
            pltpu.SemaphoreType.DMA,
            pltpu.SemaphoreType.DMA,
        ],
    )
    def scatter(feat_hbm, idx_hbm, out_hbm, cnt_hbm, sums, cnts, idxb, v0b, v1b,
                sem0, sem1):
        wid = lax.axis_index("s") * NC + lax.axis_index("c")
        c0 = wid * cpw
        zero = jnp.zeros((L,), jnp.float32)
        ones = jnp.ones((L,), jnp.float32)
        roff = jnp.full((L,), R, jnp.int32)
        sems = (sem0, sem1)

        for b in range(B):
            @plsc.parallel_loop(0, (cpw * R) // L, unroll=8)
            def zsums(i):
                sums[pl.ds(i * L, L)] = zero

            @pl.when(wid == b)
            def _zc():
                @plsc.parallel_loop(0, R // L, unroll=8)
                def zcnts(i):
                    cnts[pl.ds(i * L, L)] = zero

            def issue(k, slot):
                so = slot * chunk
                pltpu.async_copy(
                    idx_hbm.at[pl.ds(b * N + k * chunk, chunk)],
                    idxb.at[pl.ds(so, chunk)], sems[slot])
                pltpu.async_copy(
                    feat_hbm.at[pl.ds((b * C + c0) * N + k * chunk, chunk)],
                    v0b.at[pl.ds(so, chunk)], sems[slot])
                pltpu.async_copy(
                    feat_hbm.at[pl.ds((b * C + c0 + 1) * N + k * chunk, chunk)],
                    v1b.at[pl.ds(so, chunk)], sems[slot])

            def drain(k, slot):
                so = slot * chunk
                pltpu.make_async_copy(
                    idx_hbm.at[pl.ds(b * N + k * chunk, chunk)],
                    idxb.at[pl.ds(so, chunk)], sems[slot]).wait()
                pltpu.make_async_copy(
                    feat_hbm.at[pl.ds((b * C + c0) * N + k * chunk, chunk)],
                    v0b.at[pl.ds(so, chunk)], sems[slot]).wait()
                pltpu.make_async_copy(
                    feat_hbm.at[pl.ds((b * C + c0 + 1) * N + k * chunk, chunk)],
                    v1b.at[pl.ds(so, chunk)], sems[slot]).wait()

            def consume(slot):
                so = slot * chunk

                @plsc.parallel_loop(0, chunk // L, unroll=8)
                def g(i):
                    iv = idxb[pl.ds(so + i * L, L)]
                    plsc.addupdate_scatter(sums, [iv], v0b[pl.ds(so + i * L, L)])
                    plsc.addupdate_scatter(sums, [iv + roff], v1b[pl.ds(so + i * L, L)])

                @pl.when(wid == b)
                def _cs():
                    @plsc.parallel_loop(0, chunk // L, unroll=8)
                    def g2(i):
                        plsc.addupdate_scatter(
                            cnts, [idxb[pl.ds(so + i * L, L)]], ones)

            issue(0, 0)

            def chunk_pair(j, _):
                k = 2 * j
                issue(k + 1, 1)
                drain(k, 0)
                consume(0)

                @pl.when(k + 2 < nchunks)
                def _():
                    issue(k + 2, 0)

                drain(k + 1, 1)
                consume(1)
                return 0

            lax.fori_loop(0, nchunks // 2, chunk_pair, 0)
            if nchunks % 2:
                drain(nchunks - 1, 0)
                consume(0)

            pltpu.sync_copy(sums.at[pl.ds(0, R)],
                            out_hbm.at[pl.ds((b * C + c0) * R, R)])
            pltpu.sync_copy(sums.at[pl.ds(R, R)],
                            out_hbm.at[pl.ds((b * C + c0 + 1) * R, R)])

            @pl.when(wid == b)
            def _cw():
                pltpu.sync_copy(cnts.at[pl.ds(0, R)],
                                cnt_hbm.at[pl.ds(b * R, R)])

    return scatter


def _div_body(sums_ref, cnt_ref, out_ref):
    recip = 1.0 / jnp.maximum(cnt_ref[...], 1.0)
    g = sums_ref[...].reshape(out_ref.shape[1], R)
    out_ref[0] = g * recip[None, :]


def kernel(features, coords):
    B, C, N = features.shape
    nc_out, flat_idx = pl.pallas_call(
        _coords_body,
        grid=(B,),
        in_specs=[pl.BlockSpec((1, 3, N), lambda b: (b, 0, 0))],
        out_specs=[
            pl.BlockSpec((1, 3, N), lambda b: (b, 0, 0)),
            pl.BlockSpec((1, 1, N), lambda b: (b, 0, 0)),
        ],
        out_shape=[
            jax.ShapeDtypeStruct((B, 3, N), jnp.float32),
            jax.ShapeDtypeStruct((B, 1, N), jnp.int32),
        ],
    )(coords)
    flat_idx = flat_idx.reshape(B, N)

    scatter = _make_scatter(B, C, N, chunk=4000)
    sums, cnt = scatter(features.reshape(B * C * N), flat_idx.reshape(B * N))

    cblk = 8
    out = pl.pallas_call(
        _div_body,
        grid=(B, C // cblk),
        in_specs=[
            pl.BlockSpec((cblk * R,), lambda b, c: (b * (C // cblk) + c,)),
            pl.BlockSpec((R,), lambda b, c: (b,)),
        ],
        out_specs=pl.BlockSpec((1, cblk, R), lambda b, c: (b, c, 0)),
        out_shape=jax.ShapeDtypeStruct((B, C, R), jnp.float32),
    )(sums, cnt)
    return out.reshape(B, C, RX, RY, RZ), nc_out
